# baseline (speedup 1.0000x reference)
import functools

import jax
import jax.numpy as jnp
from jax import lax
from jax.experimental import pallas as pl
from jax.experimental.pallas import tpu as pltpu
from jax.experimental.pallas import tpu_sc as plsc

N = 10000
E = 160000
D = 128
H = 512
H2 = H // 2

NCORE = 2
NSUB = 16
NPAD = 10240
RPS = NPAD // NSUB

NW = NCORE * NSUB
GW = 40
SW = 128
ECHUNKS = ((0, 51200), (51200, 51200), (102400, 57600))

BLK_N = 1000
BLK_E = 1600


def _pack_bf16_pair(even_f32, odd_f32):
    e16 = jax.lax.bitcast_convert_type(even_f32.astype(jnp.bfloat16), jnp.uint16)
    o16 = jax.lax.bitcast_convert_type(odd_f32.astype(jnp.bfloat16), jnp.uint16)
    word = e16.astype(jnp.uint32) | (o16.astype(jnp.uint32) << 16)
    return jax.lax.bitcast_convert_type(word, jnp.int32)


def _unpack_bf16_pair(word_i32):
    u = jax.lax.bitcast_convert_type(word_i32, jnp.uint32)
    e16 = (u & jnp.uint32(0xFFFF)).astype(jnp.uint16)
    o16 = (u >> 16).astype(jnp.uint16)
    e = jax.lax.bitcast_convert_type(e16, jnp.bfloat16).astype(jnp.float32)
    o = jax.lax.bitcast_convert_type(o16, jnp.bfloat16).astype(jnp.float32)
    return e, o


def _sc_mesh():
    return plsc.VectorSubcoreMesh(core_axis_name="c", subcore_axis_name="s")


def _sc_gather(P, Q, src, dst, e_lo, ne):
    share = ne // NW
    gpw = share // GW
    assert share % 8 == 0 and gpw * GW == share

    @functools.partial(
        pl.kernel,
        out_type=jax.ShapeDtypeStruct((ne, H2), jnp.int32),
        mesh=_sc_mesh(),
        compiler_params=pltpu.CompilerParams(needs_layout_passes=False),
        scratch_types=[
            pltpu.VMEM((share,), jnp.int32),
            pltpu.VMEM((share,), jnp.int32),
            pltpu.VMEM((2, GW, H2), jnp.int32),
            pltpu.VMEM((2, GW, H2), jnp.int32),
            pltpu.SemaphoreType.DMA,
            pltpu.SemaphoreType.DMA,
            pltpu.SemaphoreType.DMA,
            pltpu.SemaphoreType.DMA,
        ],
    )
    def k(p_hbm, q_hbm, src_hbm, dst_hbm, s_hbm,
          sidx, didx, bp2, bq2, sg0, sg1, sw0, sw1):
        cid = lax.axis_index("c")
        sid = lax.axis_index("s")
        wbase = (sid * NCORE + cid) * share
        pltpu.sync_copy(src_hbm.at[pl.ds(e_lo + wbase, share)], sidx)
        pltpu.sync_copy(dst_hbm.at[pl.ds(e_lo + wbase, share)], didx)
        sg = (sg0, sg1)
        sw = (sw0, sw1)

        def issue_gathers(jj, b):
            e0 = jj * GW
            pltpu.async_copy(q_hbm.at[sidx.at[pl.ds(e0, GW)]], bq2.at[b], sg[b])
            pltpu.async_copy(p_hbm.at[didx.at[pl.ds(e0, GW)]], bp2.at[b], sg[b])

        def wait_gathers(b, rows):
            pltpu.make_async_copy(q_hbm.at[rows], bq2.at[b], sg[b]).wait()
            pltpu.make_async_copy(p_hbm.at[rows], bp2.at[b], sg[b]).wait()

        def add_pq(b):
            bq = bq2.at[b]
            bp = bp2.at[b]

            @pl.loop(0, GW)
            def _(r):
                for c in range(H2 // 16):
                    sl = pl.ds(c * 16, 16)
                    qv = plsc.bitcast(bq[r, sl], jnp.bfloat16)
                    pv = plsc.bitcast(bp[r, sl], jnp.bfloat16)
                    bq[r, sl] = plsc.bitcast(qv + pv, jnp.int32)

        def issue_write(jj, b):
            rows = pl.ds(wbase + jj * GW, GW)
            pltpu.async_copy(bq2.at[b], s_hbm.at[rows], sw[b])

        def wait_write(b, rows):
            pltpu.make_async_copy(bq2.at[b], s_hbm.at[rows], sw[b]).wait()

        def step(jj, b):
            rows = pl.ds(wbase + jj * GW, GW)

            @pl.when(jj >= 2)
            def _():
                wait_write(b, rows)

            issue_gathers(jj, b)

            @pl.when(jj >= 1)
            def _():
                prev = pl.ds(wbase + (jj - 1) * GW, GW)
                wait_gathers(1 - b, prev)
                add_pq(1 - b)
                issue_write(jj - 1, 1 - b)

        if gpw % 2 == 0:
            @pl.loop(0, gpw - 1, step=2)
            def _(j):
                for b in range(2):
                    step(j + b, b)
        else:
            @pl.loop(0, gpw - 2, step=2)
            def _(j):
                for b in range(2):
                    step(j + b, b)

            step(gpw - 1, (gpw - 1) % 2)

        last = gpw - 1
        pb = last % 2
        rows = pl.ds(wbase + last * GW, GW)
        prev = pl.ds(wbase + (last - 1) * GW, GW)
        wait_gathers(pb, rows)
        add_pq(pb)
        issue_write(last, pb)
        wait_write(1 - pb, prev)
        wait_write(pb, rows)

    return k(P, Q, src, dst)


def _sc_scatter(m, dst, init, e_lo, ne):
    nsch = ne // SW
    off = e_lo // SW

    @functools.partial(
        pl.kernel,
        out_type=jax.ShapeDtypeStruct((NCORE, NPAD, D), jnp.float32),
        mesh=_sc_mesh(),
        scratch_types=[pltpu.VMEM_SHARED((NPAD, D), jnp.float32)],
    )
    def k(m_hbm, dst_hbm, init_hbm, out_hbm, acc):
        cid = lax.axis_index("c")
        sid = lax.axis_index("s")
        r0 = sid * RPS
        pltpu.sync_copy(init_hbm.at[cid, pl.ds(r0, RPS)], acc.at[pl.ds(r0, RPS)])
        plsc.subcore_barrier()

        def body(m_v, div):
            pltpu.sync_copy(m_v, acc.at[div.at[0]], add=True)

        pltpu.emit_pipeline(
            body,
            grid=(nsch,),
            in_specs=[
                pl.BlockSpec((SW, D), lambda i: (i, 0)),
                pl.BlockSpec((1, SW), lambda i: (0, i + off)),
            ],
            out_specs=[],
            core_axis_name=("c", "s"),
            dimension_semantics=(pltpu.PARALLEL,),
        )(m_hbm, dst_hbm)

        plsc.subcore_barrier()
        pltpu.sync_copy(acc.at[pl.ds(r0, RPS)], out_hbm.at[cid, pl.ds(r0, RPS)])

    return k(m, dst, init)


def _sc_count(dst, ones_w, zeros_nd):
    @functools.partial(
        pl.kernel,
        out_type=jax.ShapeDtypeStruct((NCORE, NPAD, D), jnp.float32),
        mesh=_sc_mesh(),
        scratch_types=[
            pltpu.VMEM_SHARED((NPAD, D), jnp.float32),
            pltpu.VMEM((SW, D), jnp.float32),
        ],
    )
    def k(dst_hbm, ones_hbm, z_hbm, out_hbm, acc, ones_v):
        cid = lax.axis_index("c")
        sid = lax.axis_index("s")
        r0 = sid * RPS
        pltpu.sync_copy(ones_hbm, ones_v)
        pltpu.sync_copy(z_hbm.at[pl.ds(r0, RPS)], acc.at[pl.ds(r0, RPS)])
        plsc.subcore_barrier()

        def body(div):
            pltpu.sync_copy(ones_v, acc.at[div.at[0]], add=True)

        pltpu.emit_pipeline(
            body,
            grid=(E // SW,),
            in_specs=[pl.BlockSpec((1, SW), lambda i: (0, i))],
            out_specs=[],
            core_axis_name=("c", "s"),
            dimension_semantics=(pltpu.PARALLEL,),
        )(dst_hbm)

        plsc.subcore_barrier()
        pltpu.sync_copy(acc.at[pl.ds(r0, RPS)], out_hbm.at[cid, pl.ds(r0, RPS)])

    return k(dst, ones_w, zeros_nd)


def _tc_node0(x, W1e, W1o, b1e, b1o):
    def body(x_ref, we_ref, wo_ref, be_ref, bo_ref, p_ref, q_ref):
        y = x_ref[...]
        pe = jnp.dot(y, we_ref[:D, :] - we_ref[D:, :],
                     preferred_element_type=jnp.float32) + be_ref[...]
        po = jnp.dot(y, wo_ref[:D, :] - wo_ref[D:, :],
                     preferred_element_type=jnp.float32) + bo_ref[...]
        p_ref[...] = _pack_bf16_pair(pe, po)
        qe = jnp.dot(y, we_ref[D:, :], preferred_element_type=jnp.float32)
        qo = jnp.dot(y, wo_ref[D:, :], preferred_element_type=jnp.float32)
        q_ref[...] = _pack_bf16_pair(qe, qo)

    return pl.pallas_call(
        body,
        grid=(N // BLK_N,),
        in_specs=[
            pl.BlockSpec((BLK_N, D), lambda i: (i, 0)),
            pl.BlockSpec((2 * D, H2), lambda i: (0, 0)),
            pl.BlockSpec((2 * D, H2), lambda i: (0, 0)),
            pl.BlockSpec((1, H2), lambda i: (0, 0)),
            pl.BlockSpec((1, H2), lambda i: (0, 0)),
        ],
        out_specs=[
            pl.BlockSpec((BLK_N, H2), lambda i: (i, 0)),
            pl.BlockSpec((BLK_N, H2), lambda i: (i, 0)),
        ],
        out_shape=[jax.ShapeDtypeStruct((N, H2), jnp.int32)] * 2,
    )(x, W1e, W1o, b1e.reshape(1, H2), b1o.reshape(1, H2))


def _tc_node_ep(parts, cntp, W1e, W1o, b1e, b1o):
    def body(pp_ref, c_ref, we_ref, wo_ref, be_ref, bo_ref, p_ref, q_ref):
        s = pp_ref[0] + pp_ref[1]
        c = c_ref[0, :, 0:1] + c_ref[1, :, 0:1]
        y = jnp.maximum(s / jnp.maximum(c, 1.0), 0.0)
        pe = jnp.dot(y, we_ref[:D, :] - we_ref[D:, :],
                     preferred_element_type=jnp.float32) + be_ref[...]
        po = jnp.dot(y, wo_ref[:D, :] - wo_ref[D:, :],
                     preferred_element_type=jnp.float32) + bo_ref[...]
        p_ref[...] = _pack_bf16_pair(pe, po)
        qe = jnp.dot(y, we_ref[D:, :], preferred_element_type=jnp.float32)
        qo = jnp.dot(y, wo_ref[D:, :], preferred_element_type=jnp.float32)
        q_ref[...] = _pack_bf16_pair(qe, qo)

    return pl.pallas_call(
        body,
        grid=(N // BLK_N,),
        in_specs=[
            pl.BlockSpec((NCORE, BLK_N, D), lambda i: (0, i, 0)),
            pl.BlockSpec((NCORE, BLK_N, D), lambda i: (0, i, 0)),
            pl.BlockSpec((2 * D, H2), lambda i: (0, 0)),
            pl.BlockSpec((2 * D, H2), lambda i: (0, 0)),
            pl.BlockSpec((1, H2), lambda i: (0, 0)),
            pl.BlockSpec((1, H2), lambda i: (0, 0)),
        ],
        out_specs=[
            pl.BlockSpec((BLK_N, H2), lambda i: (i, 0)),
            pl.BlockSpec((BLK_N, H2), lambda i: (i, 0)),
        ],
        out_shape=[jax.ShapeDtypeStruct((N, H2), jnp.int32)] * 2,
    )(parts, cntp, W1e, W1o, b1e.reshape(1, H2), b1o.reshape(1, H2))


def _tc_edge(S, W2e, W2o, b2, ne):
    def body(s_ref, w2e_ref, w2o_ref, b2_ref, m_ref):
        he, ho = _unpack_bf16_pair(s_ref[...])
        he = jnp.maximum(he, 0.0)
        ho = jnp.maximum(ho, 0.0)
        m_ref[...] = (
            jnp.dot(he, w2e_ref[...], preferred_element_type=jnp.float32)
            + jnp.dot(ho, w2o_ref[...], preferred_element_type=jnp.float32)
            + b2_ref[...]
        )

    return pl.pallas_call(
        body,
        grid=(ne // BLK_E,),
        in_specs=[
            pl.BlockSpec((BLK_E, H2), lambda i: (i, 0)),
            pl.BlockSpec((H2, D), lambda i: (0, 0)),
            pl.BlockSpec((H2, D), lambda i: (0, 0)),
            pl.BlockSpec((1, D), lambda i: (0, 0)),
        ],
        out_specs=pl.BlockSpec((BLK_E, D), lambda i: (i, 0)),
        out_shape=jax.ShapeDtypeStruct((ne, D), jnp.float32),
    )(S, W2e, W2o, b2.reshape(1, D))


def _tc_final(parts, cntp):
    def body(pp_ref, c_ref, o_ref):
        s = pp_ref[0] + pp_ref[1]
        c = c_ref[0, :, 0:1] + c_ref[1, :, 0:1]
        o_ref[...] = s / jnp.maximum(c, 1.0)

    return pl.pallas_call(
        body,
        grid=(N // BLK_N,),
        in_specs=[
            pl.BlockSpec((NCORE, BLK_N, D), lambda i: (0, i, 0)),
            pl.BlockSpec((NCORE, BLK_N, D), lambda i: (0, i, 0)),
        ],
        out_specs=pl.BlockSpec((BLK_N, D), lambda i: (i, 0)),
        out_shape=jax.ShapeDtypeStruct((N, D), jnp.float32),
    )(parts, cntp)


def kernel(x, edge_index, W1_0, b1_0, W2_0, b2_0, W1_1, b1_1, W2_1, b2_1,
           W1_2, b1_2, W2_2, b2_2):
    src1 = edge_index[0]
    dst1 = edge_index[1]
    dst = dst1.reshape(1, E)
    zeros_nd = jnp.zeros((NPAD, D), jnp.float32)
    zeros_parts = jnp.zeros((NCORE, NPAD, D), jnp.float32)
    ones_w = jnp.ones((SW, D), jnp.float32)

    cntp = _sc_count(dst, ones_w, zeros_nd)

    parts = None
    for l, (W1, b1, W2, b2) in enumerate(
        [(W1_0, b1_0, W2_0, b2_0), (W1_1, b1_1, W2_1, b2_1),
         (W1_2, b1_2, W2_2, b2_2)]
    ):
        W1e, W1o = W1[:, 0::2], W1[:, 1::2]
        b1e, b1o = b1[0::2], b1[1::2]
        W2e, W2o = W2[0::2, :], W2[1::2, :]
        if l == 0:
            P, Q = _tc_node0(x, W1e, W1o, b1e, b1o)
        else:
            P, Q = _tc_node_ep(parts, cntp, W1e, W1o, b1e, b1o)
        ms = []
        for e_lo, ne in ECHUNKS:
            S = _sc_gather(P, Q, src1, dst1, e_lo, ne)
            ms.append(_tc_edge(S, W2e, W2o, b2, ne))
        parts = zeros_parts
        for (e_lo, ne), m in zip(ECHUNKS, ms):
            parts = _sc_scatter(m, dst, parts, e_lo, ne)

    return _tc_final(parts, cntp)


# speedup vs baseline: 1.0072x; 1.0072x over previous
import functools

import jax
import jax.numpy as jnp
from jax import lax
from jax.experimental import pallas as pl
from jax.experimental.pallas import tpu as pltpu
from jax.experimental.pallas import tpu_sc as plsc

N = 10000
E = 160000
D = 128
H = 512
H2 = H // 2

NCORE = 2
NSUB = 16
NPAD = 10240
RPS = NPAD // NSUB

NW = NCORE * NSUB
GW = 40
SW = 128
ECHUNKS = ((0, 32000), (32000, 128000))

BLK_N = 1000
BLK_E = 2000


def _pack_bf16_pair(even_f32, odd_f32):
    e16 = jax.lax.bitcast_convert_type(even_f32.astype(jnp.bfloat16), jnp.uint16)
    o16 = jax.lax.bitcast_convert_type(odd_f32.astype(jnp.bfloat16), jnp.uint16)
    word = e16.astype(jnp.uint32) | (o16.astype(jnp.uint32) << 16)
    return jax.lax.bitcast_convert_type(word, jnp.int32)


def _unpack_bf16_pair(word_i32):
    u = jax.lax.bitcast_convert_type(word_i32, jnp.uint32)
    e16 = (u & jnp.uint32(0xFFFF)).astype(jnp.uint16)
    o16 = (u >> 16).astype(jnp.uint16)
    e = jax.lax.bitcast_convert_type(e16, jnp.bfloat16).astype(jnp.float32)
    o = jax.lax.bitcast_convert_type(o16, jnp.bfloat16).astype(jnp.float32)
    return e, o


def _sc_mesh():
    return plsc.VectorSubcoreMesh(core_axis_name="c", subcore_axis_name="s")


def _sc_gather(P, Q, src, dst, e_lo, ne):
    share = ne // NW
    gpw = share // GW
    assert share % 8 == 0 and gpw * GW == share

    @functools.partial(
        pl.kernel,
        out_type=jax.ShapeDtypeStruct((ne, H2), jnp.int32),
        mesh=_sc_mesh(),
        compiler_params=pltpu.CompilerParams(needs_layout_passes=False),
        scratch_types=[
            pltpu.VMEM((share,), jnp.int32),
            pltpu.VMEM((share,), jnp.int32),
            pltpu.VMEM((2, GW, H2), jnp.int32),
            pltpu.VMEM((2, GW, H2), jnp.int32),
            pltpu.SemaphoreType.DMA,
            pltpu.SemaphoreType.DMA,
            pltpu.SemaphoreType.DMA,
            pltpu.SemaphoreType.DMA,
        ],
    )
    def k(p_hbm, q_hbm, src_hbm, dst_hbm, s_hbm,
          sidx, didx, bp2, bq2, sg0, sg1, sw0, sw1):
        cid = lax.axis_index("c")
        sid = lax.axis_index("s")
        wbase = (sid * NCORE + cid) * share
        pltpu.sync_copy(src_hbm.at[pl.ds(e_lo + wbase, share)], sidx)
        pltpu.sync_copy(dst_hbm.at[pl.ds(e_lo + wbase, share)], didx)
        sg = (sg0, sg1)
        sw = (sw0, sw1)

        def issue_gathers(jj, b):
            e0 = jj * GW
            pltpu.async_copy(q_hbm.at[sidx.at[pl.ds(e0, GW)]], bq2.at[b], sg[b])
            pltpu.async_copy(p_hbm.at[didx.at[pl.ds(e0, GW)]], bp2.at[b], sg[b])

        def wait_gathers(b, rows):
            pltpu.make_async_copy(q_hbm.at[rows], bq2.at[b], sg[b]).wait()
            pltpu.make_async_copy(p_hbm.at[rows], bp2.at[b], sg[b]).wait()

        def add_pq(b):
            bq = bq2.at[b]
            bp = bp2.at[b]

            @pl.loop(0, GW)
            def _(r):
                for c in range(H2 // 16):
                    sl = pl.ds(c * 16, 16)
                    qv = plsc.bitcast(bq[r, sl], jnp.bfloat16)
                    pv = plsc.bitcast(bp[r, sl], jnp.bfloat16)
                    bq[r, sl] = plsc.bitcast(qv + pv, jnp.int32)

        def issue_write(jj, b):
            rows = pl.ds(wbase + jj * GW, GW)
            pltpu.async_copy(bq2.at[b], s_hbm.at[rows], sw[b])

        def wait_write(b, rows):
            pltpu.make_async_copy(bq2.at[b], s_hbm.at[rows], sw[b]).wait()

        def step(jj, b):
            rows = pl.ds(wbase + jj * GW, GW)

            @pl.when(jj >= 2)
            def _():
                wait_write(b, rows)

            issue_gathers(jj, b)

            @pl.when(jj >= 1)
            def _():
                prev = pl.ds(wbase + (jj - 1) * GW, GW)
                wait_gathers(1 - b, prev)
                add_pq(1 - b)
                issue_write(jj - 1, 1 - b)

        if gpw % 2 == 0:
            @pl.loop(0, gpw - 1, step=2)
            def _(j):
                for b in range(2):
                    step(j + b, b)
        else:
            @pl.loop(0, gpw - 2, step=2)
            def _(j):
                for b in range(2):
                    step(j + b, b)

            step(gpw - 1, (gpw - 1) % 2)

        last = gpw - 1
        pb = last % 2
        rows = pl.ds(wbase + last * GW, GW)
        prev = pl.ds(wbase + (last - 1) * GW, GW)
        wait_gathers(pb, rows)
        add_pq(pb)
        issue_write(last, pb)
        wait_write(1 - pb, prev)
        wait_write(pb, rows)

    return k(P, Q, src, dst)


def _sc_scatter(m, dst, init, e_lo, ne):
    nsch = ne // SW
    off = e_lo // SW

    @functools.partial(
        pl.kernel,
        out_type=jax.ShapeDtypeStruct((NCORE, NPAD, D), jnp.float32),
        mesh=_sc_mesh(),
        scratch_types=[pltpu.VMEM_SHARED((NPAD, D), jnp.float32)],
    )
    def k(m_hbm, dst_hbm, init_hbm, out_hbm, acc):
        cid = lax.axis_index("c")
        sid = lax.axis_index("s")
        r0 = sid * RPS
        pltpu.sync_copy(init_hbm.at[cid, pl.ds(r0, RPS)], acc.at[pl.ds(r0, RPS)])
        plsc.subcore_barrier()

        def body(m_v, div):
            pltpu.sync_copy(m_v, acc.at[div.at[0]], add=True)

        pltpu.emit_pipeline(
            body,
            grid=(nsch,),
            in_specs=[
                pl.BlockSpec((SW, D), lambda i: (i, 0)),
                pl.BlockSpec((1, SW), lambda i: (0, i + off)),
            ],
            out_specs=[],
            core_axis_name=("c", "s"),
            dimension_semantics=(pltpu.PARALLEL,),
        )(m_hbm, dst_hbm)

        plsc.subcore_barrier()
        pltpu.sync_copy(acc.at[pl.ds(r0, RPS)], out_hbm.at[cid, pl.ds(r0, RPS)])

    return k(m, dst, init)


def _sc_count(dst, ones_w, zeros_nd):
    @functools.partial(
        pl.kernel,
        out_type=jax.ShapeDtypeStruct((NCORE, NPAD, D), jnp.float32),
        mesh=_sc_mesh(),
        scratch_types=[
            pltpu.VMEM_SHARED((NPAD, D), jnp.float32),
            pltpu.VMEM((SW, D), jnp.float32),
        ],
    )
    def k(dst_hbm, ones_hbm, z_hbm, out_hbm, acc, ones_v):
        cid = lax.axis_index("c")
        sid = lax.axis_index("s")
        r0 = sid * RPS
        pltpu.sync_copy(ones_hbm, ones_v)
        pltpu.sync_copy(z_hbm.at[pl.ds(r0, RPS)], acc.at[pl.ds(r0, RPS)])
        plsc.subcore_barrier()

        def body(div):
            pltpu.sync_copy(ones_v, acc.at[div.at[0]], add=True)

        pltpu.emit_pipeline(
            body,
            grid=(E // SW,),
            in_specs=[pl.BlockSpec((1, SW), lambda i: (0, i))],
            out_specs=[],
            core_axis_name=("c", "s"),
            dimension_semantics=(pltpu.PARALLEL,),
        )(dst_hbm)

        plsc.subcore_barrier()
        pltpu.sync_copy(acc.at[pl.ds(r0, RPS)], out_hbm.at[cid, pl.ds(r0, RPS)])

    return k(dst, ones_w, zeros_nd)


def _tc_node0(x, W1e, W1o, b1e, b1o):
    def body(x_ref, we_ref, wo_ref, be_ref, bo_ref, p_ref, q_ref):
        y = x_ref[...]
        pe = jnp.dot(y, we_ref[:D, :] - we_ref[D:, :],
                     preferred_element_type=jnp.float32) + be_ref[...]
        po = jnp.dot(y, wo_ref[:D, :] - wo_ref[D:, :],
                     preferred_element_type=jnp.float32) + bo_ref[...]
        p_ref[...] = _pack_bf16_pair(pe, po)
        qe = jnp.dot(y, we_ref[D:, :], preferred_element_type=jnp.float32)
        qo = jnp.dot(y, wo_ref[D:, :], preferred_element_type=jnp.float32)
        q_ref[...] = _pack_bf16_pair(qe, qo)

    return pl.pallas_call(
        body,
        grid=(N // BLK_N,),
        in_specs=[
            pl.BlockSpec((BLK_N, D), lambda i: (i, 0)),
            pl.BlockSpec((2 * D, H2), lambda i: (0, 0)),
            pl.BlockSpec((2 * D, H2), lambda i: (0, 0)),
            pl.BlockSpec((1, H2), lambda i: (0, 0)),
            pl.BlockSpec((1, H2), lambda i: (0, 0)),
        ],
        out_specs=[
            pl.BlockSpec((BLK_N, H2), lambda i: (i, 0)),
            pl.BlockSpec((BLK_N, H2), lambda i: (i, 0)),
        ],
        out_shape=[jax.ShapeDtypeStruct((N, H2), jnp.int32)] * 2,
    )(x, W1e, W1o, b1e.reshape(1, H2), b1o.reshape(1, H2))


def _tc_node_ep(parts, cntp, W1e, W1o, b1e, b1o):
    def body(pp_ref, c_ref, we_ref, wo_ref, be_ref, bo_ref, p_ref, q_ref):
        s = pp_ref[0] + pp_ref[1]
        c = c_ref[0, :, 0:1] + c_ref[1, :, 0:1]
        y = jnp.maximum(s / jnp.maximum(c, 1.0), 0.0)
        pe = jnp.dot(y, we_ref[:D, :] - we_ref[D:, :],
                     preferred_element_type=jnp.float32) + be_ref[...]
        po = jnp.dot(y, wo_ref[:D, :] - wo_ref[D:, :],
                     preferred_element_type=jnp.float32) + bo_ref[...]
        p_ref[...] = _pack_bf16_pair(pe, po)
        qe = jnp.dot(y, we_ref[D:, :], preferred_element_type=jnp.float32)
        qo = jnp.dot(y, wo_ref[D:, :], preferred_element_type=jnp.float32)
        q_ref[...] = _pack_bf16_pair(qe, qo)

    return pl.pallas_call(
        body,
        grid=(N // BLK_N,),
        in_specs=[
            pl.BlockSpec((NCORE, BLK_N, D), lambda i: (0, i, 0)),
            pl.BlockSpec((NCORE, BLK_N, D), lambda i: (0, i, 0)),
            pl.BlockSpec((2 * D, H2), lambda i: (0, 0)),
            pl.BlockSpec((2 * D, H2), lambda i: (0, 0)),
            pl.BlockSpec((1, H2), lambda i: (0, 0)),
            pl.BlockSpec((1, H2), lambda i: (0, 0)),
        ],
        out_specs=[
            pl.BlockSpec((BLK_N, H2), lambda i: (i, 0)),
            pl.BlockSpec((BLK_N, H2), lambda i: (i, 0)),
        ],
        out_shape=[jax.ShapeDtypeStruct((N, H2), jnp.int32)] * 2,
    )(parts, cntp, W1e, W1o, b1e.reshape(1, H2), b1o.reshape(1, H2))


def _tc_edge(S, W2e, W2o, b2, ne):
    def body(s_ref, w2e_ref, w2o_ref, b2_ref, m_ref):
        he, ho = _unpack_bf16_pair(s_ref[...])
        he = jnp.maximum(he, 0.0)
        ho = jnp.maximum(ho, 0.0)
        m_ref[...] = (
            jnp.dot(he, w2e_ref[...], preferred_element_type=jnp.float32)
            + jnp.dot(ho, w2o_ref[...], preferred_element_type=jnp.float32)
            + b2_ref[...]
        )

    return pl.pallas_call(
        body,
        grid=(ne // BLK_E,),
        in_specs=[
            pl.BlockSpec((BLK_E, H2), lambda i: (i, 0)),
            pl.BlockSpec((H2, D), lambda i: (0, 0)),
            pl.BlockSpec((H2, D), lambda i: (0, 0)),
            pl.BlockSpec((1, D), lambda i: (0, 0)),
        ],
        out_specs=pl.BlockSpec((BLK_E, D), lambda i: (i, 0)),
        out_shape=jax.ShapeDtypeStruct((ne, D), jnp.float32),
    )(S, W2e, W2o, b2.reshape(1, D))


def _tc_final(parts, cntp):
    def body(pp_ref, c_ref, o_ref):
        s = pp_ref[0] + pp_ref[1]
        c = c_ref[0, :, 0:1] + c_ref[1, :, 0:1]
        o_ref[...] = s / jnp.maximum(c, 1.0)

    return pl.pallas_call(
        body,
        grid=(N // BLK_N,),
        in_specs=[
            pl.BlockSpec((NCORE, BLK_N, D), lambda i: (0, i, 0)),
            pl.BlockSpec((NCORE, BLK_N, D), lambda i: (0, i, 0)),
        ],
        out_specs=pl.BlockSpec((BLK_N, D), lambda i: (i, 0)),
        out_shape=jax.ShapeDtypeStruct((N, D), jnp.float32),
    )(parts, cntp)


def kernel(x, edge_index, W1_0, b1_0, W2_0, b2_0, W1_1, b1_1, W2_1, b2_1,
           W1_2, b1_2, W2_2, b2_2):
    src1 = edge_index[0]
    dst1 = edge_index[1]
    dst = dst1.reshape(1, E)
    zeros_nd = jnp.zeros((NPAD, D), jnp.float32)
    zeros_parts = jnp.zeros((NCORE, NPAD, D), jnp.float32)
    ones_w = jnp.ones((SW, D), jnp.float32)

    cntp = _sc_count(dst, ones_w, zeros_nd)

    parts = None
    for l, (W1, b1, W2, b2) in enumerate(
        [(W1_0, b1_0, W2_0, b2_0), (W1_1, b1_1, W2_1, b2_1),
         (W1_2, b1_2, W2_2, b2_2)]
    ):
        W1e, W1o = W1[:, 0::2], W1[:, 1::2]
        b1e, b1o = b1[0::2], b1[1::2]
        W2e, W2o = W2[0::2, :], W2[1::2, :]
        if l == 0:
            P, Q = _tc_node0(x, W1e, W1o, b1e, b1o)
        else:
            P, Q = _tc_node_ep(parts, cntp, W1e, W1o, b1e, b1o)
        ms = []
        for e_lo, ne in ECHUNKS:
            S = _sc_gather(P, Q, src1, dst1, e_lo, ne)
            ms.append(_tc_edge(S, W2e, W2o, b2, ne))
        parts = zeros_parts
        for (e_lo, ne), m in zip(ECHUNKS, ms):
            parts = _sc_scatter(m, dst, parts, e_lo, ne)

    return _tc_final(parts, cntp)


# speedup vs baseline: 1.0833x; 1.0756x over previous
import functools

import jax
import jax.numpy as jnp
from jax import lax
from jax.experimental import pallas as pl
from jax.experimental.pallas import tpu as pltpu
from jax.experimental.pallas import tpu_sc as plsc

N = 10000
E = 160000
D = 128
H = 512
H2 = H // 2

NCORE = 2
NSUB = 16
NPAD = 10240
RPS = NPAD // NSUB

NW = NCORE * NSUB
GW = 40
SW = 128
ECHUNKS = ((0, 64000), (64000, 96000))

BLK_N = 1000
BLK_E = 2000


def _pack_bf16_pair(even_f32, odd_f32):
    e16 = jax.lax.bitcast_convert_type(even_f32.astype(jnp.bfloat16), jnp.uint16)
    o16 = jax.lax.bitcast_convert_type(odd_f32.astype(jnp.bfloat16), jnp.uint16)
    word = e16.astype(jnp.uint32) | (o16.astype(jnp.uint32) << 16)
    return jax.lax.bitcast_convert_type(word, jnp.int32)


def _unpack_bf16_pair(word_i32):
    u = jax.lax.bitcast_convert_type(word_i32, jnp.uint32)
    e16 = (u & jnp.uint32(0xFFFF)).astype(jnp.uint16)
    o16 = (u >> 16).astype(jnp.uint16)
    e = jax.lax.bitcast_convert_type(e16, jnp.bfloat16).astype(jnp.float32)
    o = jax.lax.bitcast_convert_type(o16, jnp.bfloat16).astype(jnp.float32)
    return e, o


def _sc_mesh():
    return plsc.VectorSubcoreMesh(core_axis_name="c", subcore_axis_name="s")


def _sc_gather(P, Q, src, dst, e_lo, ne):
    share = ne // NW
    gpw = share // GW
    assert share % 8 == 0 and gpw * GW == share

    @functools.partial(
        pl.kernel,
        out_type=jax.ShapeDtypeStruct((ne, H2), jnp.int32),
        mesh=_sc_mesh(),
        compiler_params=pltpu.CompilerParams(needs_layout_passes=False),
        scratch_types=[
            pltpu.VMEM((share,), jnp.int32),
            pltpu.VMEM((share,), jnp.int32),
            pltpu.VMEM((3, GW, H2), jnp.int32),
            pltpu.VMEM((3, GW, H2), jnp.int32),
            pltpu.SemaphoreType.DMA,
            pltpu.SemaphoreType.DMA,
            pltpu.SemaphoreType.DMA,
            pltpu.SemaphoreType.DMA,
            pltpu.SemaphoreType.DMA,
            pltpu.SemaphoreType.DMA,
        ],
    )
    def k(p_hbm, q_hbm, src_hbm, dst_hbm, s_hbm,
          sidx, didx, bp2, bq2, sg0, sg1, sg2, sw0, sw1, sw2):
        cid = lax.axis_index("c")
        sid = lax.axis_index("s")
        wbase = (sid * NCORE + cid) * share
        pltpu.sync_copy(src_hbm.at[pl.ds(e_lo + wbase, share)], sidx)
        pltpu.sync_copy(dst_hbm.at[pl.ds(e_lo + wbase, share)], didx)
        sg = (sg0, sg1, sg2)
        sw = (sw0, sw1, sw2)

        def issue_gathers(jj, b):
            e0 = jj * GW
            pltpu.async_copy(q_hbm.at[sidx.at[pl.ds(e0, GW)]], bq2.at[b], sg[b])
            pltpu.async_copy(p_hbm.at[didx.at[pl.ds(e0, GW)]], bp2.at[b], sg[b])

        def wait_gathers(b, rows):
            pltpu.make_async_copy(q_hbm.at[rows], bq2.at[b], sg[b]).wait()
            pltpu.make_async_copy(p_hbm.at[rows], bp2.at[b], sg[b]).wait()

        def add_pq(b):
            bq = bq2.at[b]
            bp = bp2.at[b]

            @pl.loop(0, GW)
            def _(r):
                for c in range(H2 // 16):
                    sl = pl.ds(c * 16, 16)
                    qv = plsc.bitcast(bq[r, sl], jnp.bfloat16)
                    pv = plsc.bitcast(bp[r, sl], jnp.bfloat16)
                    bq[r, sl] = plsc.bitcast(qv + pv, jnp.int32)

        def issue_write(jj, b):
            rows = pl.ds(wbase + jj * GW, GW)
            pltpu.async_copy(bq2.at[b], s_hbm.at[rows], sw[b])

        def wait_write(b, rows):
            pltpu.make_async_copy(bq2.at[b], s_hbm.at[rows], sw[b]).wait()

        def retire(t, tb):
            trows = pl.ds(wbase + t * GW, GW)
            wait_gathers(tb, trows)
            add_pq(tb)
            issue_write(t, tb)

        def step(jj, b):
            rows = pl.ds(wbase + jj * GW, GW)

            @pl.when(jj >= 3)
            def _():
                wait_write(b, rows)

            issue_gathers(jj, b)

            @pl.when(jj >= 2)
            def _():
                retire(jj - 2, (b + 1) % 3)

        loop_n = (gpw // 3) * 3
        @pl.loop(0, loop_n, step=3)
        def _(j):
            for b in range(3):
                step(j + b, b)

        for jj in range(loop_n, gpw):
            step(jj, jj % 3)

        for t in range(max(gpw - 2, 0), gpw):
            retire(t, t % 3)
        for t in range(max(gpw - 3, 0), gpw):
            wait_write(t % 3, pl.ds(wbase + t * GW, GW))

    return k(P, Q, src, dst)


def _sc_scatter(m, dst, init, e_lo, ne):
    nsch = ne // SW
    off = e_lo // SW

    @functools.partial(
        pl.kernel,
        out_type=jax.ShapeDtypeStruct((NCORE, NPAD, D), jnp.float32),
        mesh=_sc_mesh(),
        scratch_types=[pltpu.VMEM_SHARED((NPAD, D), jnp.float32)],
    )
    def k(m_hbm, dst_hbm, init_hbm, out_hbm, acc):
        cid = lax.axis_index("c")
        sid = lax.axis_index("s")
        r0 = sid * RPS
        pltpu.sync_copy(init_hbm.at[cid, pl.ds(r0, RPS)], acc.at[pl.ds(r0, RPS)])
        plsc.subcore_barrier()

        def body(m_v, div):
            pltpu.sync_copy(m_v, acc.at[div.at[0]], add=True)

        pltpu.emit_pipeline(
            body,
            grid=(nsch,),
            in_specs=[
                pl.BlockSpec((SW, D), lambda i: (i, 0)),
                pl.BlockSpec((1, SW), lambda i: (0, i + off)),
            ],
            out_specs=[],
            core_axis_name=("c", "s"),
            dimension_semantics=(pltpu.PARALLEL,),
        )(m_hbm, dst_hbm)

        plsc.subcore_barrier()
        pltpu.sync_copy(acc.at[pl.ds(r0, RPS)], out_hbm.at[cid, pl.ds(r0, RPS)])

    return k(m, dst, init)


def _sc_count(dst, ones_w, zeros_nd):
    @functools.partial(
        pl.kernel,
        out_type=jax.ShapeDtypeStruct((NCORE, NPAD, D), jnp.float32),
        mesh=_sc_mesh(),
        scratch_types=[
            pltpu.VMEM_SHARED((NPAD, D), jnp.float32),
            pltpu.VMEM((SW, D), jnp.float32),
        ],
    )
    def k(dst_hbm, ones_hbm, z_hbm, out_hbm, acc, ones_v):
        cid = lax.axis_index("c")
        sid = lax.axis_index("s")
        r0 = sid * RPS
        pltpu.sync_copy(ones_hbm, ones_v)
        pltpu.sync_copy(z_hbm.at[pl.ds(r0, RPS)], acc.at[pl.ds(r0, RPS)])
        plsc.subcore_barrier()

        def body(div):
            pltpu.sync_copy(ones_v, acc.at[div.at[0]], add=True)

        pltpu.emit_pipeline(
            body,
            grid=(E // SW,),
            in_specs=[pl.BlockSpec((1, SW), lambda i: (0, i))],
            out_specs=[],
            core_axis_name=("c", "s"),
            dimension_semantics=(pltpu.PARALLEL,),
        )(dst_hbm)

        plsc.subcore_barrier()
        pltpu.sync_copy(acc.at[pl.ds(r0, RPS)], out_hbm.at[cid, pl.ds(r0, RPS)])

    return k(dst, ones_w, zeros_nd)


def _tc_node0(x, W1e, W1o, b1e, b1o):
    def body(x_ref, we_ref, wo_ref, be_ref, bo_ref, p_ref, q_ref):
        y = x_ref[...]
        pe = jnp.dot(y, we_ref[:D, :] - we_ref[D:, :],
                     preferred_element_type=jnp.float32) + be_ref[...]
        po = jnp.dot(y, wo_ref[:D, :] - wo_ref[D:, :],
                     preferred_element_type=jnp.float32) + bo_ref[...]
        p_ref[...] = _pack_bf16_pair(pe, po)
        qe = jnp.dot(y, we_ref[D:, :], preferred_element_type=jnp.float32)
        qo = jnp.dot(y, wo_ref[D:, :], preferred_element_type=jnp.float32)
        q_ref[...] = _pack_bf16_pair(qe, qo)

    return pl.pallas_call(
        body,
        grid=(N // BLK_N,),
        in_specs=[
            pl.BlockSpec((BLK_N, D), lambda i: (i, 0)),
            pl.BlockSpec((2 * D, H2), lambda i: (0, 0)),
            pl.BlockSpec((2 * D, H2), lambda i: (0, 0)),
            pl.BlockSpec((1, H2), lambda i: (0, 0)),
            pl.BlockSpec((1, H2), lambda i: (0, 0)),
        ],
        out_specs=[
            pl.BlockSpec((BLK_N, H2), lambda i: (i, 0)),
            pl.BlockSpec((BLK_N, H2), lambda i: (i, 0)),
        ],
        out_shape=[jax.ShapeDtypeStruct((N, H2), jnp.int32)] * 2,
    )(x, W1e, W1o, b1e.reshape(1, H2), b1o.reshape(1, H2))


def _tc_node_ep(parts, cntp, W1e, W1o, b1e, b1o):
    def body(pp_ref, c_ref, we_ref, wo_ref, be_ref, bo_ref, p_ref, q_ref):
        s = pp_ref[0] + pp_ref[1]
        c = c_ref[0, :, 0:1] + c_ref[1, :, 0:1]
        y = jnp.maximum(s / jnp.maximum(c, 1.0), 0.0)
        pe = jnp.dot(y, we_ref[:D, :] - we_ref[D:, :],
                     preferred_element_type=jnp.float32) + be_ref[...]
        po = jnp.dot(y, wo_ref[:D, :] - wo_ref[D:, :],
                     preferred_element_type=jnp.float32) + bo_ref[...]
        p_ref[...] = _pack_bf16_pair(pe, po)
        qe = jnp.dot(y, we_ref[D:, :], preferred_element_type=jnp.float32)
        qo = jnp.dot(y, wo_ref[D:, :], preferred_element_type=jnp.float32)
        q_ref[...] = _pack_bf16_pair(qe, qo)

    return pl.pallas_call(
        body,
        grid=(N // BLK_N,),
        in_specs=[
            pl.BlockSpec((NCORE, BLK_N, D), lambda i: (0, i, 0)),
            pl.BlockSpec((NCORE, BLK_N, D), lambda i: (0, i, 0)),
            pl.BlockSpec((2 * D, H2), lambda i: (0, 0)),
            pl.BlockSpec((2 * D, H2), lambda i: (0, 0)),
            pl.BlockSpec((1, H2), lambda i: (0, 0)),
            pl.BlockSpec((1, H2), lambda i: (0, 0)),
        ],
        out_specs=[
            pl.BlockSpec((BLK_N, H2), lambda i: (i, 0)),
            pl.BlockSpec((BLK_N, H2), lambda i: (i, 0)),
        ],
        out_shape=[jax.ShapeDtypeStruct((N, H2), jnp.int32)] * 2,
    )(parts, cntp, W1e, W1o, b1e.reshape(1, H2), b1o.reshape(1, H2))


def _tc_edge(S, W2e, W2o, b2, ne):
    def body(s_ref, w2e_ref, w2o_ref, b2_ref, m_ref):
        he, ho = _unpack_bf16_pair(s_ref[...])
        he = jnp.maximum(he, 0.0)
        ho = jnp.maximum(ho, 0.0)
        m_ref[...] = (
            jnp.dot(he, w2e_ref[...], preferred_element_type=jnp.float32)
            + jnp.dot(ho, w2o_ref[...], preferred_element_type=jnp.float32)
            + b2_ref[...]
        )

    return pl.pallas_call(
        body,
        grid=(ne // BLK_E,),
        in_specs=[
            pl.BlockSpec((BLK_E, H2), lambda i: (i, 0)),
            pl.BlockSpec((H2, D), lambda i: (0, 0)),
            pl.BlockSpec((H2, D), lambda i: (0, 0)),
            pl.BlockSpec((1, D), lambda i: (0, 0)),
        ],
        out_specs=pl.BlockSpec((BLK_E, D), lambda i: (i, 0)),
        out_shape=jax.ShapeDtypeStruct((ne, D), jnp.float32),
    )(S, W2e, W2o, b2.reshape(1, D))


def _tc_final(parts, cntp):
    def body(pp_ref, c_ref, o_ref):
        s = pp_ref[0] + pp_ref[1]
        c = c_ref[0, :, 0:1] + c_ref[1, :, 0:1]
        o_ref[...] = s / jnp.maximum(c, 1.0)

    return pl.pallas_call(
        body,
        grid=(N // BLK_N,),
        in_specs=[
            pl.BlockSpec((NCORE, BLK_N, D), lambda i: (0, i, 0)),
            pl.BlockSpec((NCORE, BLK_N, D), lambda i: (0, i, 0)),
        ],
        out_specs=pl.BlockSpec((BLK_N, D), lambda i: (i, 0)),
        out_shape=jax.ShapeDtypeStruct((N, D), jnp.float32),
    )(parts, cntp)


def kernel(x, edge_index, W1_0, b1_0, W2_0, b2_0, W1_1, b1_1, W2_1, b2_1,
           W1_2, b1_2, W2_2, b2_2):
    src1 = edge_index[0]
    dst1 = edge_index[1]
    dst = dst1.reshape(1, E)
    zeros_nd = jnp.zeros((NPAD, D), jnp.float32)
    zeros_parts = jnp.zeros((NCORE, NPAD, D), jnp.float32)
    ones_w = jnp.ones((SW, D), jnp.float32)

    cntp = _sc_count(dst, ones_w, zeros_nd)

    parts = None
    for l, (W1, b1, W2, b2) in enumerate(
        [(W1_0, b1_0, W2_0, b2_0), (W1_1, b1_1, W2_1, b2_1),
         (W1_2, b1_2, W2_2, b2_2)]
    ):
        W1e, W1o = W1[:, 0::2], W1[:, 1::2]
        b1e, b1o = b1[0::2], b1[1::2]
        W2e, W2o = W2[0::2, :], W2[1::2, :]
        if l == 0:
            P, Q = _tc_node0(x, W1e, W1o, b1e, b1o)
        else:
            P, Q = _tc_node_ep(parts, cntp, W1e, W1o, b1e, b1o)
        ms = []
        for e_lo, ne in ECHUNKS:
            S = _sc_gather(P, Q, src1, dst1, e_lo, ne)
            ms.append(_tc_edge(S, W2e, W2o, b2, ne))
        parts = zeros_parts
        for (e_lo, ne), m in zip(ECHUNKS, ms):
            parts = _sc_scatter(m, dst, parts, e_lo, ne)

    return _tc_final(parts, cntp)
